# baseline (speedup 1.0000x reference)
import functools

import jax
import jax.numpy as jnp
from jax import lax
from jax.experimental import pallas as pl
from jax.experimental.pallas import tpu as pltpu
from jax.experimental.pallas import tpu_sc as plsc

_EMB = 1000000
_D = 64
_B = 16384
_K = 5

_NC = 2
_NS = 16
_NW = _NC * _NS
_BPW = _B // _NW
_CH = 128
_UCH = _BPW // _CH

_mesh = plsc.VectorSubcoreMesh(core_axis_name="c", subcore_axis_name="s")


def _worker_base():
    c = lax.axis_index("c")
    s = lax.axis_index("s")
    return (s * _NC + c) * _BPW


def _chunk_seq(table, idx_of, nch, out, obase, rows, sems):
    def fire(j, slot):
        def body(g, carry):
            vec = idx_of(j, g)
            for k in range(16):
                pltpu.async_copy(
                    table.at[vec[k]], rows.at[slot].at[g * 16 + k],
                    sems[slot])
            return carry
        lax.fori_loop(0, _CH // 16, body, 0)

    def drain(slot):
        pltpu.make_async_copy(
            out.at[pl.ds(0, _CH)], rows.at[slot], sems[slot]).wait()

    fire(0, 0)
    for j in range(nch):
        slot = j % 2
        if j + 1 < nch:
            fire(j + 1, 1 - slot)
        drain(slot)
        pltpu.sync_copy(rows.at[slot], out.at[pl.ds(obase + j * _CH, _CH)])


@functools.partial(
    pl.kernel,
    mesh=_mesh,
    compiler_params=pltpu.CompilerParams(use_tc_tiling_on_sc=False),
    out_type=[jax.ShapeDtypeStruct((_B, _D), jnp.float32)],
    scratch_types=[
        pltpu.VMEM((_BPW,), jnp.int32),
        pltpu.VMEM((2, _CH, _D), jnp.float32),
        pltpu.SemaphoreType.DMA,
        pltpu.SemaphoreType.DMA,
    ],
)
def _sc_gather_u(pos_u, u_embs, out_u, idx_u, rows, sem0, sem1):
    base = _worker_base()
    pltpu.sync_copy(pos_u.at[pl.ds(base, _BPW)], idx_u)
    _chunk_seq(u_embs, lambda j, g: idx_u[pl.ds(j * _CH + g * 16, 16)],
               _UCH, out_u, base, rows, (sem0, sem1))


@functools.partial(
    pl.kernel,
    mesh=_mesh,
    out_type=[
        jax.ShapeDtypeStruct((_B, _D), jnp.float32),
        jax.ShapeDtypeStruct((_K * _B, _D), jnp.float32),
    ],
    scratch_types=[
        pltpu.VMEM((_BPW,), jnp.int32),
        pltpu.VMEM((_K, _BPW), jnp.int32),
        pltpu.VMEM((2, _CH, _D), jnp.float32),
        pltpu.SemaphoreType.DMA,
        pltpu.SemaphoreType.DMA,
    ],
)
def _sc_gather_vn(pos_v, neg_vt, v_embs, out_v, out_n,
                  idx_v, idx_n, rows, sem0, sem1):
    base = _worker_base()
    sems = (sem0, sem1)
    pltpu.sync_copy(pos_v.at[pl.ds(base, _BPW)], idx_v)
    pltpu.sync_copy(neg_vt.at[:, pl.ds(base, _BPW)], idx_n)
    _chunk_seq(v_embs, lambda j, g: idx_v[pl.ds(j * _CH + g * 16, 16)],
               _UCH, out_v, base, rows, sems)
    for k in range(_K):
        _chunk_seq(v_embs,
                   lambda j, g, _k=k: idx_n[_k, pl.ds(j * _CH + g * 16, 16)],
                   _UCH, out_n, k * _B + base, rows, sems)


_BLK = 1024
_G = _B // _BLK


def _tc_loss_body(u_ref, v_ref, n_ref, out_ref):
    u = u_ref[...]
    v = v_ref[...]
    n = n_ref[...]
    score = jnp.sum(u * v, axis=1)
    score = jnp.clip(score, -10.0, 10.0)
    pos_l = jnp.log1p(jnp.exp(-score))
    ns = jnp.sum(n * u[None, :, :], axis=-1)
    ns = jnp.clip(ns, -10.0, 10.0)
    neg_l = jnp.sum(jnp.log1p(jnp.exp(ns)), axis=0)
    inc = (jnp.sum(pos_l + neg_l) * (1.0 / _B))[None, None]

    @pl.when(pl.program_id(0) == 0)
    def _():
        out_ref[...] = jnp.zeros((1, 1), jnp.float32)

    out_ref[...] += inc


_tc_loss = pl.pallas_call(
    _tc_loss_body,
    grid=(_G,),
    in_specs=[
        pl.BlockSpec((_BLK, _D), lambda i: (i, 0)),
        pl.BlockSpec((_BLK, _D), lambda i: (i, 0)),
        pl.BlockSpec((_K, _BLK, _D), lambda i: (0, i, 0)),
    ],
    out_specs=pl.BlockSpec((1, 1), lambda i: (0, 0)),
    out_shape=jax.ShapeDtypeStruct((1, 1), jnp.float32),
)


def kernel(pos_u, pos_v, neg_v, u_embs, v_embs):
    (rows_u,) = _sc_gather_u(pos_u.astype(jnp.int32), u_embs)
    rows_v, rows_n = _sc_gather_vn(
        pos_v.astype(jnp.int32), neg_v.T.astype(jnp.int32), v_embs)
    out = _tc_loss(rows_u, rows_v, rows_n.reshape(_K, _B, _D))
    return out[0, 0]


# speedup vs baseline: 1.3642x; 1.3642x over previous
import functools

import jax
import jax.numpy as jnp
from jax import lax
from jax.experimental import pallas as pl
from jax.experimental.pallas import tpu as pltpu
from jax.experimental.pallas import tpu_sc as plsc

_EMB = 1000000
_D = 64
_B = 16384
_K = 5

_NC = 2
_NS = 16
_NW = _NC * _NS
_BPW = _B // _NW
_CH = 128
_UCH = _BPW // _CH

_mesh = plsc.VectorSubcoreMesh(core_axis_name="c", subcore_axis_name="s")


def _worker_base():
    c = lax.axis_index("c")
    s = lax.axis_index("s")
    return (s * _NC + c) * _BPW


def _chunk_seq(table, idx_of, nch, out, obase, rows, sems):
    def fire(j, slot):
        def body(g, carry):
            vec = idx_of(j, g)
            for k in range(16):
                pltpu.async_copy(
                    table.at[vec[k]], rows.at[slot].at[g * 16 + k],
                    sems[slot])
            return carry
        lax.fori_loop(0, _CH // 16, body, 0)

    def drain(slot):
        pltpu.make_async_copy(
            out.at[pl.ds(0, _CH)], rows.at[slot], sems[slot]).wait()

    fire(0, 0)
    for j in range(nch):
        slot = j % 2
        if j + 1 < nch:
            fire(j + 1, 1 - slot)
        drain(slot)
        pltpu.sync_copy(rows.at[slot], out.at[pl.ds(obase + j * _CH, _CH)])


@functools.partial(
    pl.kernel,
    mesh=_mesh,
    out_type=[jax.ShapeDtypeStruct((_B, _D), jnp.float32)],
    scratch_types=[
        pltpu.VMEM((_BPW,), jnp.int32),
        pltpu.VMEM((2, _CH, _D), jnp.float32),
        pltpu.SemaphoreType.DMA,
        pltpu.SemaphoreType.DMA,
    ],
)
def _sc_gather_u(pos_u, u_embs, out_u, idx_u, rows, sem0, sem1):
    base = _worker_base()
    pltpu.sync_copy(pos_u.at[pl.ds(base, _BPW)], idx_u)
    _chunk_seq(u_embs, lambda j, g: idx_u[pl.ds(j * _CH + g * 16, 16)],
               _UCH, out_u, base, rows, (sem0, sem1))


@functools.partial(
    pl.kernel,
    mesh=_mesh,
    out_type=[
        jax.ShapeDtypeStruct((_B, _D), jnp.float32),
        jax.ShapeDtypeStruct((_K * _B, _D), jnp.float32),
    ],
    scratch_types=[
        pltpu.VMEM((_BPW,), jnp.int32),
        pltpu.VMEM((_K, _BPW), jnp.int32),
        pltpu.VMEM((2, _CH, _D), jnp.float32),
        pltpu.SemaphoreType.DMA,
        pltpu.SemaphoreType.DMA,
    ],
)
def _sc_gather_vn(pos_v, neg_vt, v_embs, out_v, out_n,
                  idx_v, idx_n, rows, sem0, sem1):
    base = _worker_base()
    sems = (sem0, sem1)
    pltpu.sync_copy(pos_v.at[pl.ds(base, _BPW)], idx_v)
    pltpu.sync_copy(neg_vt.at[:, pl.ds(base, _BPW)], idx_n)
    _chunk_seq(v_embs, lambda j, g: idx_v[pl.ds(j * _CH + g * 16, 16)],
               _UCH, out_v, base, rows, sems)
    for k in range(_K):
        _chunk_seq(v_embs,
                   lambda j, g, _k=k: idx_n[_k, pl.ds(j * _CH + g * 16, 16)],
                   _UCH, out_n, k * _B + base, rows, sems)


_BLK = 1024
_G = _B // _BLK


def _tc_loss_body(u_ref, v_ref, n_ref, out_ref):
    u = u_ref[...]
    v = v_ref[...]
    n = n_ref[...]
    score = jnp.sum(u * v, axis=1)
    score = jnp.clip(score, -10.0, 10.0)
    pos_l = jnp.log1p(jnp.exp(-score))
    ns = jnp.sum(n * u[None, :, :], axis=-1)
    ns = jnp.clip(ns, -10.0, 10.0)
    neg_l = jnp.sum(jnp.log1p(jnp.exp(ns)), axis=0)
    inc = (jnp.sum(pos_l + neg_l) * (1.0 / _B))[None, None]

    @pl.when(pl.program_id(0) == 0)
    def _():
        out_ref[...] = jnp.zeros((1, 1), jnp.float32)

    out_ref[...] += inc


_tc_loss = pl.pallas_call(
    _tc_loss_body,
    grid=(_G,),
    in_specs=[
        pl.BlockSpec((_BLK, _D), lambda i: (i, 0)),
        pl.BlockSpec((_BLK, _D), lambda i: (i, 0)),
        pl.BlockSpec((_K, _BLK, _D), lambda i: (0, i, 0)),
    ],
    out_specs=pl.BlockSpec((1, 1), lambda i: (0, 0)),
    out_shape=jax.ShapeDtypeStruct((1, 1), jnp.float32),
)


def kernel(pos_u, pos_v, neg_v, u_embs, v_embs):
    (rows_u,) = _sc_gather_u(pos_u.astype(jnp.int32), u_embs)
    rows_v, rows_n = _sc_gather_vn(
        pos_v.astype(jnp.int32), neg_v.T.astype(jnp.int32), v_embs)
    out = _tc_loss(rows_u, rows_v, rows_n.reshape(_K, _B, _D))
    return out[0, 0]
